# baseline (speedup 1.0000x reference)
import functools

import jax
import jax.numpy as jnp
from jax import lax
from jax.experimental import pallas as pl
from jax.experimental.pallas import tpu as pltpu
from jax.experimental.pallas import tpu_sc as plsc

IY = 225
IX = 224
IYB = (IY // 128) * 128
B, H, W, C = 32, 450, 449, 24
NCH = 6


def _sc_point_gather(xt, chan16):
    mesh = plsc.VectorSubcoreMesh(
        core_axis_name="c", subcore_axis_name="s", num_cores=1
    )

    @functools.partial(
        pl.kernel,
        mesh=mesh,
        out_type=jax.ShapeDtypeStruct((NCH, B), jnp.float32),
        scratch_types=[
            pltpu.VMEM((B, C, 128), jnp.float32),
            pltpu.VMEM((16,), jnp.int32),
            pltpu.VMEM((NCH, B), jnp.float32),
        ],
        compiler_params=pltpu.CompilerParams(needs_layout_passes=False),
    )
    def k(xt_hbm, chan_hbm, out_hbm, point_v, chan_v, out_v):
        wid = lax.axis_index("s") + lax.axis_index("c")

        @pl.when(wid == 0)
        def _():
            pltpu.sync_copy(xt_hbm.at[:, IX, :, pl.ds(IYB, 128)], point_v)
            pltpu.sync_copy(chan_hbm, chan_v.at[pl.ds(0, NCH)])
            lanes = lax.iota(jnp.int32, 16)
            six = jnp.full((16,), NCH, jnp.int32)
            yoff = jnp.full((16,), IY - IYB, jnp.int32)
            for g in range(B * NCH // 16):
                f = lanes + g * 16
                b = lax.div(f, six)
                jj = lax.rem(f, six)
                c = plsc.load_gather(chan_v, [jj])
                vals = plsc.load_gather(point_v, [b, c, yoff])
                plsc.store_scatter(out_v, [jj, b], vals)
            pltpu.sync_copy(out_v, out_hbm)

    return k(xt, chan16)


def kernel(x, chan):
    xt = x.transpose(0, 2, 3, 1)
    return _sc_point_gather(xt, chan.astype(jnp.int32)).T


# speedup vs baseline: 1.1799x; 1.1799x over previous
import functools

import jax
import jax.numpy as jnp
from jax import lax
from jax.experimental import pallas as pl
from jax.experimental.pallas import tpu as pltpu
from jax.experimental.pallas import tpu_sc as plsc

IY = 225
IX = 224
IYB = (IY // 128) * 128
B, H, W, C = 32, 450, 449, 24
NCH = 6
BPT = 2
NT = B // BPT
OPT = BPT * NCH


def _sc_point_gather(xt, chan):
    mesh = plsc.VectorSubcoreMesh(
        core_axis_name="c", subcore_axis_name="s", num_cores=1
    )

    @functools.partial(
        pl.kernel,
        mesh=mesh,
        out_type=jax.ShapeDtypeStruct((NCH, B), jnp.float32),
        scratch_types=[
            pltpu.VMEM((BPT, C, 128), jnp.float32),
            pltpu.VMEM((16,), jnp.int32),
            pltpu.VMEM((16,), jnp.float32),
            pltpu.VMEM_SHARED((NT, 16), jnp.float32),
            pltpu.VMEM((NT, 16), jnp.float32),
            pltpu.VMEM((NCH, B), jnp.float32),
        ],
        compiler_params=pltpu.CompilerParams(needs_layout_passes=False),
    )
    def k(xt_hbm, chan_hbm, out_hbm, win_v, chan_v, row_v, shared, asm_v, out_v):
        s = lax.axis_index("s")
        lanes = lax.iota(jnp.int32, 16)
        six = jnp.full((16,), NCH, jnp.int32)
        pltpu.sync_copy(
            xt_hbm.at[pl.ds(s * BPT, BPT), IX, :, pl.ds(IYB, 128)], win_v
        )
        pltpu.sync_copy(chan_hbm, chan_v.at[pl.ds(0, NCH)])
        b = jnp.minimum(lax.div(lanes, six), jnp.full((16,), BPT - 1, jnp.int32))
        c = plsc.load_gather(chan_v, [lax.rem(lanes, six)])
        yoff = jnp.full((16,), IY - IYB, jnp.int32)
        row_v[...] = plsc.load_gather(win_v, [b, c, yoff])
        pltpu.sync_copy(row_v, shared.at[s])
        plsc.subcore_barrier()

        @pl.when(s == 0)
        def _():
            pltpu.sync_copy(shared, asm_v)
            two = jnp.full((16,), BPT, jnp.int32)
            for j in range(NCH):
                for h in range(2):
                    bb = lanes + h * 16
                    r = lax.div(bb, two)
                    col = lax.rem(bb, two) * six + j
                    out_v[j, pl.ds(h * 16, 16)] = plsc.load_gather(
                        asm_v, [r, col]
                    )
            pltpu.sync_copy(out_v, out_hbm)

    return k(xt, chan)


def kernel(x, chan):
    xt = x.transpose(0, 2, 3, 1)
    return _sc_point_gather(xt, chan.astype(jnp.int32)).T


# speedup vs baseline: 1.1839x; 1.0034x over previous
import functools

import jax
import jax.numpy as jnp
from jax import lax
from jax.experimental import pallas as pl
from jax.experimental.pallas import tpu as pltpu
from jax.experimental.pallas import tpu_sc as plsc

IY = 225
IX = 224
IYB = (IY // 128) * 128
B, H, W, C = 32, 450, 449, 24
NCH = 6
BPT = 2
NT = B // BPT
OPT = BPT * NCH


def _sc_point_gather(xt, chan):
    mesh = plsc.VectorSubcoreMesh(
        core_axis_name="c", subcore_axis_name="s", num_cores=1
    )

    @functools.partial(
        pl.kernel,
        mesh=mesh,
        out_type=jax.ShapeDtypeStruct((NCH, B), jnp.float32),
        scratch_types=[
            pltpu.VMEM((BPT, C, 128), jnp.float32),
            pltpu.VMEM((16,), jnp.int32),
            pltpu.VMEM((16,), jnp.float32),
            pltpu.VMEM_SHARED((NT * 16,), jnp.float32),
            pltpu.VMEM((NT * 16,), jnp.float32),
            pltpu.VMEM((NCH, B), jnp.float32),
        ],
        compiler_params=pltpu.CompilerParams(needs_layout_passes=False),
    )
    def k(xt_hbm, chan_hbm, out_hbm, win_v, chan_v, row_v, shared, asm_v, out_v):
        s = lax.axis_index("s")
        lanes = lax.iota(jnp.int32, 16)
        six = jnp.full((16,), NCH, jnp.int32)
        pltpu.sync_copy(
            xt_hbm.at[pl.ds(s * BPT, BPT), IX, :, pl.ds(IYB, 128)], win_v
        )
        pltpu.sync_copy(chan_hbm, chan_v.at[pl.ds(0, NCH)])
        b = jnp.minimum(lax.div(lanes, six), jnp.full((16,), BPT - 1, jnp.int32))
        c = plsc.load_gather(chan_v, [lax.rem(lanes, six)])
        yoff = jnp.full((16,), IY - IYB, jnp.int32)
        row_v[...] = plsc.load_gather(win_v, [b, c, yoff])
        pltpu.sync_copy(row_v, shared.at[pl.ds(s * 16, 16)])
        plsc.subcore_barrier()

        @pl.when(s == 0)
        def _():
            pltpu.sync_copy(shared, asm_v)
            two = jnp.full((16,), BPT, jnp.int32)
            sixteen = jnp.full((16,), 16, jnp.int32)
            for j in range(NCH):
                for h in range(2):
                    bb = lanes + h * 16
                    idx = lax.div(bb, two) * sixteen + lax.rem(bb, two) * six + j
                    out_v[j, pl.ds(h * 16, 16)] = plsc.load_gather(asm_v, [idx])
            pltpu.sync_copy(out_v, out_hbm)

    return k(xt, chan)


def kernel(x, chan):
    xt = x.transpose(0, 2, 3, 1)
    return _sc_point_gather(xt, chan.astype(jnp.int32)).T
